# baseline (speedup 1.0000x reference)
import jax
import jax.numpy as jnp
from jax.experimental import pallas as pl
from jax.experimental.pallas import tpu as pltpu

NUM_EMBEDDINGS = 1024
EMBEDDING_DIM = 64
TOKENS = 16 * 32 * 32
BLOCK_TOKENS = 2048
NUM_BLOCKS = TOKENS // BLOCK_TOKENS


def _vq_block(z_ref, emb_ref, quant_ref, idx_ref):
    z = z_ref[...]
    emb = emb_ref[...]
    e_sq = jnp.sum(emb * emb, axis=1, keepdims=True)
    z_sq = jnp.sum(z * z, axis=1, keepdims=True).T
    ndot = jax.lax.dot_general(
        emb, -2.0 * z, (((1,), (1,)), ((), ())),
        preferred_element_type=jnp.float32)
    dist = (z_sq + e_sq) + ndot
    minv = jnp.min(dist, axis=0, keepdims=True)
    iota_col = jax.lax.broadcasted_iota(jnp.int32, (NUM_EMBEDDINGS, 1), 0
                                        ).astype(jnp.float32)
    idx_f = jnp.min(jnp.where(dist == minv, iota_col, float(NUM_EMBEDDINGS)),
                    axis=0)
    idx_i = idx_f.astype(jnp.int32)
    idx_ref[0, 0, :] = idx_i
    onehot_t = (jax.lax.broadcasted_iota(jnp.int32, (NUM_EMBEDDINGS, 1), 0)
                == idx_i[None, :]).astype(jnp.bfloat16)
    quant_ref[...] = jax.lax.dot_general(
        onehot_t, emb, (((0,), (0,)), ((), ())),
        preferred_element_type=jnp.float32)


def _vq_tc(flat, embedding):
    return pl.pallas_call(
        _vq_block,
        grid=(NUM_BLOCKS,),
        compiler_params=pltpu.CompilerParams(
            dimension_semantics=("parallel",)),
        in_specs=[
            pl.BlockSpec((BLOCK_TOKENS, EMBEDDING_DIM), lambda b: (b, 0)),
            pl.BlockSpec((NUM_EMBEDDINGS, EMBEDDING_DIM), lambda b: (0, 0)),
        ],
        out_specs=[
            pl.BlockSpec((BLOCK_TOKENS, EMBEDDING_DIM), lambda b: (b, 0)),
            pl.BlockSpec((1, 1, BLOCK_TOKENS), lambda b: (b, 0, 0)),
        ],
        out_shape=[
            jax.ShapeDtypeStruct((TOKENS, EMBEDDING_DIM), jnp.float32),
            jax.ShapeDtypeStruct((NUM_BLOCKS, 1, BLOCK_TOKENS), jnp.int32),
        ],
    )(flat, embedding)


def kernel(hidden_states, embedding):
    flat = hidden_states.reshape(TOKENS, EMBEDDING_DIM)
    quant, idx = _vq_tc(flat, embedding)

    z_q = quant.reshape(hidden_states.shape)
    B = hidden_states.shape[0]
    min_encoding_indices = idx.reshape(B, TOKENS // B)
    return (z_q, min_encoding_indices)
